# baseline (speedup 1.0000x reference)
import functools

import jax
import jax.numpy as jnp
from jax import lax
from jax.experimental import pallas as pl
from jax.experimental.pallas import tpu as pltpu

_ALPHA = 0.2


def _gat_body(nheads, nhid, nclass, bi, ni, x_ref, w3_ref, as_ref,
              adj_ref, w2_ref, aout_ref, out_ref,
              haug_s, fsrc_s, fdstT_s, h2aug_s, f2_s, f2T_s, adjb_s):
    k = pl.program_id(0)
    w = nhid + 1
    alpha = jnp.bfloat16(_ALPHA)

    @pl.when(k == 0)
    def _prep():
        w_all = jnp.concatenate([w3_ref[hd] for hd in range(nheads)], axis=1)
        h = jnp.dot(x_ref[...], w_all, preferred_element_type=jnp.float32)
        fsrc_cols = []
        fdst_cols = []
        for hd in range(nheads):
            arow = as_ref[hd:hd + 1, :]
            hh = h[:, hd * nhid:(hd + 1) * nhid]
            fsrc_cols.append(jnp.dot(
                hh, jnp.transpose(arow[:, 0:nhid]),
                preferred_element_type=jnp.float32))
            fdst_cols.append(jnp.dot(
                hh, jnp.transpose(arow[:, nhid:]),
                preferred_element_type=jnp.float32))
        fsrc_s[...] = (-jnp.concatenate(fsrc_cols, axis=1)).astype(jnp.bfloat16)
        fdstT_s[...] = jnp.transpose(
            -jnp.concatenate(fdst_cols, axis=1)).astype(jnp.bfloat16)
        ones = jnp.ones((h.shape[0], 1), jnp.bfloat16)
        for hd in range(nheads):
            haug_s[:, hd * w:hd * w + nhid] = (
                h[:, hd * nhid:(hd + 1) * nhid].astype(jnp.bfloat16))
            haug_s[:, hd * w + nhid:hd * w + nhid + 1] = ones

    @pl.when(k < ni)
    def _layer1():
        adjb = adj_ref[...].astype(jnp.bfloat16)
        adjb_s[pl.ds(k * bi, bi), :] = adjb
        haug = haug_s[...]
        fsrc = fsrc_s[pl.ds(k * bi, bi), :]
        parts = []
        for hd in range(nheads):
            s = fsrc[:, hd:hd + 1] + fdstT_s[hd:hd + 1, :]
            m = jnp.minimum(s, alpha * s)
            p = jnp.exp(m) * adjb
            parts.append(jnp.dot(p, haug[:, hd * w:(hd + 1) * w],
                                 preferred_element_type=jnp.float32))
        cols = []
        for hd in range(nheads):
            hp = parts[hd][:, 0:nhid]
            rs = parts[hd][:, nhid:nhid + 1]
            x = hp / rs
            cols.append(jnp.where(x > 0.0, x, jnp.exp(x) - 1.0))
        x1 = jnp.concatenate(cols, axis=1)
        h2 = jnp.dot(x1, w2_ref[...], preferred_element_type=jnp.float32)
        a2m = -jnp.concatenate(
            [jnp.transpose(aout_ref[:, 0:nclass]),
             jnp.transpose(aout_ref[:, nclass:])], axis=1)
        f2 = jnp.dot(h2, a2m, preferred_element_type=jnp.float32)
        f2_s[pl.ds(k * bi, bi), :] = f2.astype(jnp.bfloat16)
        f2T_s[k] = jnp.transpose(f2).astype(jnp.bfloat16)
        h2aug_s[pl.ds(k * bi, bi), 0:nclass] = h2.astype(jnp.bfloat16)
        h2aug_s[pl.ds(k * bi, bi), nclass:nclass + 1] = jnp.ones(
            (h2.shape[0], 1), jnp.bfloat16)

    @pl.when(k >= ni)
    def _layer2():
        b = k - ni
        adjb = adjb_s[pl.ds(b * bi, bi), :]
        f2dT = jnp.concatenate([f2T_s[blk] for blk in range(ni)], axis=1)
        s = f2_s[pl.ds(b * bi, bi), 0:1] + f2dT[1:2, :]
        m = jnp.minimum(s, alpha * s)
        p = jnp.exp(m) * adjb
        acc = jnp.dot(p, h2aug_s[...], preferred_element_type=jnp.float32)
        x = acc[:, 0:nclass] / acc[:, nclass:nclass + 1]
        x = jnp.where(x > 0.0, x, jnp.exp(x) - 1.0)
        mx = jnp.max(x, axis=1, keepdims=True)
        lse = mx + jnp.log(jnp.sum(jnp.exp(x - mx), axis=1, keepdims=True))
        out_ref[...] = x - lse


def kernel(features, adj, Ws, As, W_out, a_out):
    n, nfeat = features.shape
    nheads, _, nhid = Ws.shape
    nclass = W_out.shape[1]
    ndim = nheads * nhid
    w1 = nhid + 1

    BI = 1024
    ni = n // BI

    As2 = As.reshape(nheads, 2 * nhid)

    out = pl.pallas_call(
        functools.partial(_gat_body, nheads, nhid, nclass, BI, ni),
        grid=(2 * ni,),
        in_specs=[
            pl.BlockSpec((n, nfeat), lambda k: (0, 0)),
            pl.BlockSpec((nheads, nfeat, nhid), lambda k: (0, 0, 0)),
            pl.BlockSpec((nheads, 2 * nhid), lambda k: (0, 0)),
            pl.BlockSpec(
                (BI, n), lambda k, ni=ni: (jnp.where(k < ni, k, ni - 1), 0)),
            pl.BlockSpec((ndim, nclass), lambda k: (0, 0)),
            pl.BlockSpec((1, 2 * nclass), lambda k: (0, 0)),
        ],
        out_specs=pl.BlockSpec(
            (BI, nclass),
            lambda k, ni=ni: (jnp.where(k < ni, 0, k - ni), 0)),
        out_shape=jax.ShapeDtypeStruct((n, nclass), jnp.float32),
        scratch_shapes=[
            pltpu.VMEM((n, nheads * w1), jnp.bfloat16),
            pltpu.VMEM((n, nheads), jnp.bfloat16),
            pltpu.VMEM((nheads, n), jnp.bfloat16),
            pltpu.VMEM((n, nclass + 1), jnp.bfloat16),
            pltpu.VMEM((n, 2), jnp.bfloat16),
            pltpu.VMEM((ni, 2, BI), jnp.bfloat16),
            pltpu.VMEM((n, n), jnp.bfloat16),
        ],
        compiler_params=pltpu.CompilerParams(
            dimension_semantics=("arbitrary",)),
    )(features, Ws, As2, adj, W_out, a_out)

    return out


# speedup vs baseline: 1.1986x; 1.1986x over previous
import functools

import jax
import jax.numpy as jnp
from jax import lax
from jax.experimental import pallas as pl
from jax.experimental.pallas import tpu as pltpu

_ALPHA = 0.2


def _gat_body(nheads, nhid, nclass, bi, ni, x_ref, w_ref, asrc_ref, adst_ref,
              adj_ref, w2_ref, a2_ref, out_ref,
              haug_s, fsrc_s, fdstT_s, h2aug_s, f2_s, f2T_s, adjb_s):
    k = pl.program_id(0)
    w = nhid + 1
    alpha = jnp.bfloat16(_ALPHA)

    @pl.when(k == 0)
    def _prep():
        h = jnp.dot(x_ref[...], w_ref[...], preferred_element_type=jnp.float32)
        fsrc_s[...] = jnp.dot(
            h, asrc_ref[...],
            preferred_element_type=jnp.float32).astype(jnp.bfloat16)
        fdstT_s[...] = jnp.transpose(
            jnp.dot(h, adst_ref[...],
                    preferred_element_type=jnp.float32)).astype(jnp.bfloat16)
        ones = jnp.ones((h.shape[0], 1), jnp.bfloat16)
        for hd in range(nheads):
            haug_s[:, hd * w:hd * w + nhid] = (
                h[:, hd * nhid:(hd + 1) * nhid].astype(jnp.bfloat16))
            haug_s[:, hd * w + nhid:hd * w + nhid + 1] = ones

    @pl.when(k < ni)
    def _layer1():
        adjb = adj_ref[...].astype(jnp.bfloat16)
        adjb_s[pl.ds(k * bi, bi), :] = adjb
        haug = haug_s[...]
        fsrc = fsrc_s[pl.ds(k * bi, bi), :]
        parts = []
        for hd in range(nheads):
            s = fsrc[:, hd:hd + 1] + fdstT_s[hd:hd + 1, :]
            m = jnp.minimum(s, alpha * s)
            p = jnp.exp(m) * adjb
            parts.append(jnp.dot(p, haug[:, hd * w:(hd + 1) * w],
                                 preferred_element_type=jnp.float32))
        cols = []
        for hd in range(nheads):
            hp = parts[hd][:, 0:nhid]
            rs = parts[hd][:, nhid:nhid + 1]
            x = hp / rs
            cols.append(jnp.where(x > 0.0, x, jnp.exp(x) - 1.0))
        x1 = jnp.concatenate(cols, axis=1)
        h2 = jnp.dot(x1, w2_ref[...], preferred_element_type=jnp.float32)
        f2 = jnp.dot(h2, a2_ref[...], preferred_element_type=jnp.float32)
        f2_s[pl.ds(k * bi, bi), :] = f2.astype(jnp.bfloat16)
        f2T_s[k] = jnp.transpose(f2).astype(jnp.bfloat16)
        h2aug_s[pl.ds(k * bi, bi), 0:nclass] = h2.astype(jnp.bfloat16)
        h2aug_s[pl.ds(k * bi, bi), nclass:nclass + 1] = jnp.ones(
            (h2.shape[0], 1), jnp.bfloat16)

    @pl.when(k >= ni)
    def _layer2():
        b = k - ni
        adjb = adjb_s[pl.ds(b * bi, bi), :]
        f2dT = jnp.concatenate([f2T_s[blk] for blk in range(ni)], axis=1)
        s = f2_s[pl.ds(b * bi, bi), 0:1] + f2dT[1:2, :]
        m = jnp.minimum(s, alpha * s)
        p = jnp.exp(m) * adjb
        acc = jnp.dot(p, h2aug_s[...], preferred_element_type=jnp.float32)
        x = acc[:, 0:nclass] / acc[:, nclass:nclass + 1]
        x = jnp.where(x > 0.0, x, jnp.exp(x) - 1.0)
        mx = jnp.max(x, axis=1, keepdims=True)
        lse = mx + jnp.log(jnp.sum(jnp.exp(x - mx), axis=1, keepdims=True))
        out_ref[...] = x - lse


def kernel(features, adj, Ws, As, W_out, a_out):
    n, nfeat = features.shape
    nheads, _, nhid = Ws.shape
    nclass = W_out.shape[1]
    ndim = nheads * nhid
    w1 = nhid + 1

    BI = 1024
    ni = n // BI

    W_all = jnp.transpose(Ws, (1, 0, 2)).reshape(nfeat, ndim)
    eye = jnp.eye(nheads, dtype=jnp.float32)
    a_src_mat = -(eye[:, None, :] * As[:, 0, :nhid][:, :, None]).reshape(ndim, nheads)
    a_dst_mat = -(eye[:, None, :] * As[:, 0, nhid:][:, :, None]).reshape(ndim, nheads)
    a2_mat = -jnp.pad(
        jnp.stack([a_out[0, :nclass], a_out[0, nclass:]], axis=1),
        ((0, 0), (0, 6)))

    out = pl.pallas_call(
        functools.partial(_gat_body, nheads, nhid, nclass, BI, ni),
        grid=(2 * ni,),
        in_specs=[
            pl.BlockSpec((n, nfeat), lambda k: (0, 0)),
            pl.BlockSpec((nfeat, ndim), lambda k: (0, 0)),
            pl.BlockSpec((ndim, nheads), lambda k: (0, 0)),
            pl.BlockSpec((ndim, nheads), lambda k: (0, 0)),
            pl.BlockSpec(
                (BI, n), lambda k, ni=ni: (jnp.where(k < ni, k, ni - 1), 0)),
            pl.BlockSpec((ndim, nclass), lambda k: (0, 0)),
            pl.BlockSpec((nclass, 8), lambda k: (0, 0)),
        ],
        out_specs=pl.BlockSpec(
            (BI, nclass),
            lambda k, ni=ni: (jnp.where(k < ni, 0, k - ni), 0)),
        out_shape=jax.ShapeDtypeStruct((n, nclass), jnp.float32),
        scratch_shapes=[
            pltpu.VMEM((n, nheads * w1), jnp.bfloat16),
            pltpu.VMEM((n, nheads), jnp.bfloat16),
            pltpu.VMEM((nheads, n), jnp.bfloat16),
            pltpu.VMEM((n, nclass + 1), jnp.bfloat16),
            pltpu.VMEM((n, 8), jnp.bfloat16),
            pltpu.VMEM((ni, nheads, BI), jnp.bfloat16),
            pltpu.VMEM((n, n), jnp.bfloat16),
        ],
        compiler_params=pltpu.CompilerParams(
            dimension_semantics=("arbitrary",)),
    )(features, W_all, a_src_mat, a_dst_mat, adj, W_out, a2_mat)

    return out
